# Initial kernel scaffold; baseline (speedup 1.0000x reference)
import jax
import jax.numpy as jnp
from jax.experimental import pallas as pl


def kernel(feature_id, feature_val, embedding_weight):
    raise NotImplementedError("write your pallas kernel here")


# speedup vs baseline: 1.4779x; 1.4779x over previous
import functools

import jax
import jax.numpy as jnp
from jax import lax
from jax.experimental import pallas as pl
from jax.experimental.pallas import tpu as pltpu
from jax.experimental.pallas import tpu_sc as plsc

_LANES = 16


def _emb_kernel_body(R, C, G, D, num_cores,
                     ids_hbm, vals_hbm, table_hbm, out_hbm,
                     idx_v, val_v, rows_v, sem):
    wid = lax.axis_index("s") * num_cores + lax.axis_index("c")
    base = wid * R

    def chunk_body(g, carry):
        off = base + g * C
        pltpu.sync_copy(ids_hbm.at[pl.ds(off, C)], idx_v)
        copies = []
        for j in range(C // G):
            copies.append(pltpu.async_copy(
                table_hbm.at[idx_v.at[pl.ds(j * G, G)]],
                rows_v.at[pl.ds(j * G, G)], sem))
        pltpu.sync_copy(vals_hbm.at[pl.ds(off, C)], val_v)
        for cp in copies:
            cp.wait()

        def row_body(ib, c2):
            i0 = ib * _LANES
            vv = val_v[pl.ds(i0, _LANES)]
            for k in range(_LANES):
                v = vv[k]
                for h in range(D // _LANES):
                    r = rows_v[i0 + k, pl.ds(h * _LANES, _LANES)]
                    rows_v[i0 + k, pl.ds(h * _LANES, _LANES)] = r * v
            return c2

        lax.fori_loop(0, C // _LANES, row_body, 0)
        pltpu.sync_copy(rows_v, out_hbm.at[pl.ds(off, C)])
        return carry

    lax.fori_loop(0, R // C, chunk_body, 0)


def kernel(feature_id, feature_val, embedding_weight):
    B, F = feature_id.shape
    V, D = embedding_weight.shape
    N = B * F
    ids = feature_id.reshape(N).astype(jnp.int32)
    vals = feature_val.reshape(N)

    info = plsc.get_sparse_core_info()
    NW = info.num_cores * info.num_subcores
    R = N // NW
    C = 512
    G = 128

    mesh = plsc.VectorSubcoreMesh(core_axis_name="c", subcore_axis_name="s")
    body = functools.partial(_emb_kernel_body, R, C, G, D, info.num_cores)
    emb = pl.kernel(
        body,
        mesh=mesh,
        compiler_params=pltpu.CompilerParams(use_tc_tiling_on_sc=False),
        out_type=jax.ShapeDtypeStruct((N, D), jnp.float32),
        scratch_types=[
            pltpu.VMEM((C,), jnp.int32),
            pltpu.VMEM((C,), jnp.float32),
            pltpu.VMEM((C, D), jnp.float32),
            pltpu.SemaphoreType.DMA,
        ],
    )
    out = emb(ids, vals, embedding_weight)
    return out.reshape(B, F, D)
